# baseline (speedup 1.0000x reference)
import functools

import jax
import jax.numpy as jnp
from jax import lax
from jax.experimental import pallas as pl
from jax.experimental.pallas import tpu as pltpu
from jax.experimental.pallas import tpu_sc as plsc

QB = 256
KBLK = 2048
NKB = 49
KPAD = KBLK * NKB
NKEYS = 100000
NROWS = 4096
NCHUNK = KPAD // 128
CHUNKV = NCHUNK // 16
CAP = 1024
WAVE = 128
TRIG = CAP - 128 - 16
NEG = -3.0e38
PADIDX = (1 << 30)


def _normalize(x):
    norm = jnp.sqrt(jnp.sum(x * x, axis=-1, keepdims=True))
    return x / jnp.maximum(norm, 1e-12)


def _mm_body(q_ref, k_ref, out_ref, mx_ref):
    kb = pl.program_id(1)
    qn = q_ref[...]
    kn = k_ref[...]
    s = jax.lax.dot_general(
        qn, kn, (((1,), (1,)), ((), ())),
        preferred_element_type=jnp.float32)
    col = kb * KBLK + jax.lax.broadcasted_iota(jnp.int32, (QB, KBLK), 1)
    s = jnp.where(col < NKEYS, s, NEG)
    out_ref[...] = s
    mx_ref[:, kb, :] = jnp.max(s.reshape(QB, KBLK // 128, 128), axis=2)


def _scores_and_maxes(q, kpad):
    nqb = q.shape[0] // QB
    return pl.pallas_call(
        _mm_body,
        grid=(nqb, NKB),
        in_specs=[
            pl.BlockSpec((QB, 128), lambda qb, kb: (qb, 0)),
            pl.BlockSpec((KBLK, 128), lambda qb, kb: (kb, 0)),
        ],
        out_specs=[
            pl.BlockSpec((QB, KBLK), lambda qb, kb: (qb, kb)),
            pl.BlockSpec((QB, NKB, KBLK // 128), lambda qb, kb: (qb, 0, 0)),
        ],
        out_shape=[
            jax.ShapeDtypeStruct((q.shape[0], KPAD), jnp.float32),
            jax.ShapeDtypeStruct((q.shape[0], NKB, KBLK // 128), jnp.float32),
        ],
        compiler_params=pltpu.CompilerParams(
            dimension_semantics=("arbitrary", "arbitrary")),
    )(q, kpad)


FLO = -1.001
FHI = 1.001
DC = 52
DCAND = 64
NBS = 16


def _iota16():
    return lax.iota(jnp.int32, 16)


def _sum16(v):
    acc = v[0]
    for l in range(1, 16):
        acc = acc + v[l]
    return acc


def _max16(v):
    acc = v[0]
    for l in range(1, 16):
        acc = jnp.maximum(acc, v[l])
    return acc


def _ones_where(m):
    return jnp.where(m, jnp.full((16,), 1, jnp.int32),
                     jnp.full((16,), 0, jnp.int32))


def _sc_topk_filter(scores2d, maxes, nrows):
    info = plsc.get_sparse_core_info()
    nc, ns = info.num_cores, info.num_subcores
    nw = nc * ns
    rpw = nrows // nw

    @functools.partial(
        pl.kernel,
        out_type=(jax.ShapeDtypeStruct((nrows, CAP), jnp.float32),
                  jax.ShapeDtypeStruct((nrows, CAP), jnp.int32)),
        mesh=plsc.VectorSubcoreMesh(core_axis_name="c", subcore_axis_name="s"),
        scratch_types=[
            pltpu.VMEM((NCHUNK,), jnp.float32),
            pltpu.VMEM((DC * 16,), jnp.int32),
            pltpu.VMEM((WAVE, 128), jnp.float32),
            pltpu.VMEM((CAP,), jnp.float32),
            pltpu.VMEM((CAP,), jnp.int32),
            pltpu.SemaphoreType.DMA,
        ],
        compiler_params=pltpu.CompilerParams(needs_layout_passes=False),
    )
    def body(scores_hbm, maxes_hbm, outs_hbm, outi_hbm,
             maxv, cidv, wave, cbs, cbi, sem):
        wid = lax.axis_index("s") * nc + lax.axis_index("c")
        row0 = wid * rpw
        iota = _iota16()

        def per_row(j, _):
            r = row0 + j
            pltpu.sync_copy(maxes_hbm.at[r], maxv)

            def count_maxv(th):
                def cbody(c, acc):
                    v = maxv[pl.ds(c * 16, 16)]
                    return acc + _ones_where(v >= th)
                return _sum16(lax.fori_loop(0, CHUNKV, cbody,
                                            jnp.zeros((16,), jnp.int32)))

            def bs_body(t, lohi):
                lo, hi = lohi
                mid = (lo + hi) * jnp.float32(0.5)
                big = count_maxv(mid) >= 100
                return (jnp.where(big, mid, lo), jnp.where(big, hi, mid))
            tau, _ = lax.fori_loop(
                0, NBS, bs_body, (jnp.float32(FLO), jnp.float32(FHI)))

            padcid = jnp.full((16,), r * NCHUNK + NCHUNK - 1, jnp.int32)
            def fillc_body(t, _):
                cidv[pl.ds(t * 16, 16)] = padcid
                return 0
            lax.fori_loop(0, DC, fillc_body, 0)

            def csc_body(c, ccnt_v):
                m = maxv[pl.ds(c * 16, 16)] >= tau
                gid = (r * NCHUNK + c * 16) + iota
                plsc.store_scatter(cidv, [ccnt_v * 16 + iota], gid, mask=m)
                return ccnt_v + _ones_where(m)
            ccnt_v = lax.fori_loop(0, CHUNKV, csc_body,
                                   jnp.zeros((16,), jnp.int32))
            maxfill = _max16(ccnt_v)
            nwav = (maxfill * 16 + (WAVE - 1)) // WAVE

            negv = jnp.full((16,), NEG, jnp.float32)
            padi = jnp.full((16,), PADIDX, jnp.int32)
            def init_body(t, _):
                cbs[pl.ds(t * 16, 16)] = negv
                cbi[pl.ds(t * 16, 16)] = padi
                return 0
            lax.fori_loop(0, DCAND, init_body, 0)

            def reselect(cnt_v, tau_c):
                def count_cand(th):
                    def cbody(d, acc):
                        v = cbs[pl.ds(d * 16, 16)]
                        return acc + _ones_where(v >= th)
                    return _sum16(lax.fori_loop(0, DCAND, cbody,
                                                jnp.zeros((16,), jnp.int32)))

                def rs_bs(t, lohi):
                    lo2, hi2 = lohi
                    mid = (lo2 + hi2) * jnp.float32(0.5)
                    big = count_cand(mid) >= 100
                    return (jnp.where(big, mid, lo2),
                            jnp.where(big, hi2, mid))
                lo2, hi2 = lax.fori_loop(
                    0, 45, rs_bs, (tau_c, jnp.float32(FHI)))
                g = count_cand(hi2)
                quota = jnp.int32(100) - g

                def comp_body(d, carry):
                    ncv, bandc = carry
                    v = cbs[pl.ds(d * 16, 16)]
                    ivv = cbi[pl.ds(d * 16, 16)]
                    m_hi = v >= hi2
                    m_band = (v >= lo2) & (v < hi2) & (bandc < quota)
                    m = (m_hi | m_band) & (ncv < DCAND - 8)
                    plsc.store_scatter(cbs, [ncv * 16 + iota], v, mask=m)
                    plsc.store_scatter(cbi, [ncv * 16 + iota], ivv, mask=m)
                    return (ncv + _ones_where(m),
                            bandc + _ones_where(m_band))
                ncv, _ = lax.fori_loop(0, DCAND, comp_body,
                                       (jnp.zeros((16,), jnp.int32),
                                        jnp.zeros((16,), jnp.int32)))

                def clr_body(d, _):
                    stale = jnp.full((16,), d, jnp.int32) >= ncv
                    cbs[pl.ds(d * 16, 16)] = jnp.where(
                        stale, negv, cbs[pl.ds(d * 16, 16)])
                    cbi[pl.ds(d * 16, 16)] = jnp.where(
                        stale, padi, cbi[pl.ds(d * 16, 16)])
                    return 0
                lax.fori_loop(0, DCAND, clr_body, 0)
                return ncv, lo2

            def wave_body(w, carry):
                cnt_v, tau_c = carry
                cnt_v, tau_c = lax.cond(
                    _max16(cnt_v) > DCAND - 40, reselect,
                    lambda cv, t: (cv, t), cnt_v, tau_c)
                pltpu.async_copy(
                    scores_hbm.at[cidv.at[pl.ds(w * WAVE, WAVE)]],
                    wave, sem).wait()

                def chunk_body(i, cnt2_v):
                    p = w * WAVE + i
                    for jj in range(8):
                        s = wave[i, pl.ds(jj * 16, 16)]
                        m = (s >= tau_c) & (cnt2_v < DCAND - 1)
                        enc = (p * 128 + jj * 16) + iota
                        plsc.store_scatter(cbs, [cnt2_v * 16 + iota], s,
                                           mask=m)
                        plsc.store_scatter(cbi, [cnt2_v * 16 + iota], enc,
                                           mask=m)
                        cnt2_v = cnt2_v + _ones_where(m)
                    return cnt2_v
                cnt_v = lax.fori_loop(0, WAVE, chunk_body, cnt_v)
                return (cnt_v, tau_c)

            lax.fori_loop(0, nwav, wave_body,
                          (jnp.zeros((16,), jnp.int32), tau))

            def dec_body(d, _):
                enc = cbi[pl.ds(d * 16, 16)]
                hole = enc >= PADIDX
                p = jnp.minimum(enc >> 7, jnp.int32(DC * 16 - 1))
                cid = plsc.load_gather(cidv, [p])
                gidx = (cid - r * NCHUNK) * 128 + (enc & 127)
                cbi[pl.ds(d * 16, 16)] = jnp.where(hole, padi, gidx)
                return 0
            lax.fori_loop(0, DCAND, dec_body, 0)

            pltpu.sync_copy(cbs, outs_hbm.at[r])
            pltpu.sync_copy(cbi, outi_hbm.at[r])
            return 0

        lax.fori_loop(0, rpw, per_row, 0)

    return body(scores2d, maxes)


MB = 256


def _merge_body(s_ref, i_ref, os_ref, oi_ref):
    s = s_ref[...]
    ci = i_ref[...]
    big = jnp.int32(0x7FFFFFFF)
    for it in range(100):
        m = jnp.max(s, axis=1)
        is_max = s == m[:, None]
        chosen = jnp.min(jnp.where(is_max, ci, big), axis=1)
        os_ref[:, it] = m
        oi_ref[:, it] = chosen
        s = jnp.where(ci == chosen[:, None], NEG, s)


def _final_topk(cs, ci):
    nrows = cs.shape[0]
    return pl.pallas_call(
        _merge_body,
        grid=(nrows // MB,),
        in_specs=[
            pl.BlockSpec((MB, CAP), lambda qb: (qb, 0)),
            pl.BlockSpec((MB, CAP), lambda qb: (qb, 0)),
        ],
        out_specs=[
            pl.BlockSpec((MB, 100), lambda qb: (qb, 0)),
            pl.BlockSpec((MB, 100), lambda qb: (qb, 0)),
        ],
        out_shape=[
            jax.ShapeDtypeStruct((nrows, 100), jnp.float32),
            jax.ShapeDtypeStruct((nrows, 100), jnp.int32),
        ],
    )(cs, ci)


NSLICE = 4


def kernel(query_embedding, movie_tag_embeddings, k):
    query_embedding = _normalize(query_embedding)
    movie_tag_embeddings = _normalize(movie_tag_embeddings)
    kpad = jnp.pad(movie_tag_embeddings, ((0, KPAD - NKEYS), (0, 0)))
    rs = NROWS // NSLICE
    out_s, out_i = [], []
    for i in range(NSLICE):
        q = lax.slice_in_dim(query_embedding, i * rs, (i + 1) * rs)
        scores, maxes = _scores_and_maxes(q, kpad)
        cs, ci = _sc_topk_filter(
            scores.reshape(rs * NCHUNK, 128), maxes.reshape(rs, NCHUNK), rs)
        s, ix = _final_topk(cs, ci)
        out_s.append(s)
        out_i.append(ix)
    return jnp.concatenate(out_s), jnp.concatenate(out_i)


# speedup vs baseline: 1.1865x; 1.1865x over previous
import functools

import jax
import jax.numpy as jnp
from jax import lax
from jax.experimental import pallas as pl
from jax.experimental.pallas import tpu as pltpu
from jax.experimental.pallas import tpu_sc as plsc

QB = 256
KBLK = 2048
NKB = 49
KPAD = KBLK * NKB
NKEYS = 100000
NROWS = 4096
NCHUNK = KPAD // 128
CHUNKV = NCHUNK // 16
CAP = 1024
WAVE = 64
TRIG = CAP - 128 - 16
NEG = -3.0e38
PADIDX = (1 << 30)


def _normalize(x):
    norm = jnp.sqrt(jnp.sum(x * x, axis=-1, keepdims=True))
    return x / jnp.maximum(norm, 1e-12)


def _mm_body(q_ref, k_ref, out_ref, mx_ref):
    kb = pl.program_id(1)
    qn = q_ref[...]
    kn = k_ref[...]
    s = jax.lax.dot_general(
        qn, kn, (((1,), (1,)), ((), ())),
        preferred_element_type=jnp.float32)
    col = kb * KBLK + jax.lax.broadcasted_iota(jnp.int32, (QB, KBLK), 1)
    s = jnp.where(col < NKEYS, s, NEG)
    out_ref[...] = s
    mx_ref[:, kb, :] = jnp.max(s.reshape(QB, KBLK // 128, 128), axis=2)


def _scores_and_maxes(q, kpad):
    nqb = q.shape[0] // QB
    return pl.pallas_call(
        _mm_body,
        grid=(nqb, NKB),
        in_specs=[
            pl.BlockSpec((QB, 128), lambda qb, kb: (qb, 0)),
            pl.BlockSpec((KBLK, 128), lambda qb, kb: (kb, 0)),
        ],
        out_specs=[
            pl.BlockSpec((QB, KBLK), lambda qb, kb: (qb, kb)),
            pl.BlockSpec((QB, NKB, KBLK // 128), lambda qb, kb: (qb, 0, 0)),
        ],
        out_shape=[
            jax.ShapeDtypeStruct((q.shape[0], KPAD), jnp.float32),
            jax.ShapeDtypeStruct((q.shape[0], NKB, KBLK // 128), jnp.float32),
        ],
        compiler_params=pltpu.CompilerParams(
            dimension_semantics=("arbitrary", "arbitrary")),
    )(q, kpad)


FLO = -1.001
FHI = 1.001
DC = 52
DCAND = 64
NBS = 20


def _iota16():
    return lax.iota(jnp.int32, 16)


def _sum16(v):
    acc = v[0]
    for l in range(1, 16):
        acc = acc + v[l]
    return acc


def _max16(v):
    acc = v[0]
    for l in range(1, 16):
        acc = jnp.maximum(acc, v[l])
    return acc


def _ones_where(m):
    return jnp.where(m, jnp.full((16,), 1, jnp.int32),
                     jnp.full((16,), 0, jnp.int32))


def _sc_topk_filter(scores2d, maxes, nrows):
    info = plsc.get_sparse_core_info()
    nc, ns = info.num_cores, info.num_subcores
    nw = nc * ns
    rpw = nrows // nw

    @functools.partial(
        pl.kernel,
        out_type=(jax.ShapeDtypeStruct((nrows, CAP), jnp.float32),
                  jax.ShapeDtypeStruct((nrows, CAP), jnp.int32)),
        mesh=plsc.VectorSubcoreMesh(core_axis_name="c", subcore_axis_name="s"),
        scratch_types=[
            pltpu.VMEM((NCHUNK,), jnp.float32),
            pltpu.VMEM((DC * 16,), jnp.int32),
            pltpu.VMEM((WAVE, 128), jnp.float32),
            pltpu.VMEM((CAP,), jnp.float32),
            pltpu.VMEM((CAP,), jnp.int32),
            pltpu.SemaphoreType.DMA,
        ],
        compiler_params=pltpu.CompilerParams(needs_layout_passes=False),
    )
    def body(scores_hbm, maxes_hbm, outs_hbm, outi_hbm,
             maxv, cidv, wave, cbs, cbi, sem):
        wid = lax.axis_index("s") * nc + lax.axis_index("c")
        row0 = wid * rpw
        iota = _iota16()

        def per_row(j, _):
            r = row0 + j
            pltpu.sync_copy(maxes_hbm.at[r], maxv)

            def count_maxv(th):
                def cbody(c, acc):
                    v = maxv[pl.ds(c * 16, 16)]
                    return acc + _ones_where(v >= th)
                return _sum16(lax.fori_loop(0, CHUNKV, cbody,
                                            jnp.zeros((16,), jnp.int32)))

            def bs_body(t, lohi):
                lo, hi = lohi
                mid = (lo + hi) * jnp.float32(0.5)
                big = count_maxv(mid) >= 100
                return (jnp.where(big, mid, lo), jnp.where(big, hi, mid))
            tau, _ = lax.fori_loop(
                0, NBS, bs_body, (jnp.float32(FLO), jnp.float32(FHI)))

            padcid = jnp.full((16,), r * NCHUNK + NCHUNK - 1, jnp.int32)
            def fillc_body(t, _):
                cidv[pl.ds(t * 16, 16)] = padcid
                return 0
            lax.fori_loop(0, DC, fillc_body, 0)

            def csc_body(c, ccnt_v):
                m = maxv[pl.ds(c * 16, 16)] >= tau
                gid = (r * NCHUNK + c * 16) + iota
                plsc.store_scatter(cidv, [ccnt_v * 16 + iota], gid, mask=m)
                return ccnt_v + _ones_where(m)
            ccnt_v = lax.fori_loop(0, CHUNKV, csc_body,
                                   jnp.zeros((16,), jnp.int32))
            maxfill = _max16(ccnt_v)
            nwav = (maxfill * 16 + (WAVE - 1)) // WAVE

            negv = jnp.full((16,), NEG, jnp.float32)
            padi = jnp.full((16,), PADIDX, jnp.int32)
            def init_body(t, _):
                cbs[pl.ds(t * 16, 16)] = negv
                cbi[pl.ds(t * 16, 16)] = padi
                return 0
            lax.fori_loop(0, DCAND, init_body, 0)

            def reselect(cnt_v, tau_c):
                def count_cand(th):
                    def cbody(d, acc):
                        v = cbs[pl.ds(d * 16, 16)]
                        return acc + _ones_where(v >= th)
                    return _sum16(lax.fori_loop(0, DCAND, cbody,
                                                jnp.zeros((16,), jnp.int32)))

                def rs_bs(t, lohi):
                    lo2, hi2 = lohi
                    mid = (lo2 + hi2) * jnp.float32(0.5)
                    big = count_cand(mid) >= 100
                    return (jnp.where(big, mid, lo2),
                            jnp.where(big, hi2, mid))
                lo2, hi2 = lax.fori_loop(
                    0, 45, rs_bs, (tau_c, jnp.float32(FHI)))
                g = count_cand(hi2)
                quota = jnp.int32(100) - g

                def comp_body(d, carry):
                    ncv, bandc = carry
                    v = cbs[pl.ds(d * 16, 16)]
                    ivv = cbi[pl.ds(d * 16, 16)]
                    m_hi = v >= hi2
                    m_band = (v >= lo2) & (v < hi2) & (bandc < quota)
                    m = (m_hi | m_band) & (ncv < DCAND - 8)
                    plsc.store_scatter(cbs, [ncv * 16 + iota], v, mask=m)
                    plsc.store_scatter(cbi, [ncv * 16 + iota], ivv, mask=m)
                    return (ncv + _ones_where(m),
                            bandc + _ones_where(m_band))
                ncv, _ = lax.fori_loop(0, DCAND, comp_body,
                                       (jnp.zeros((16,), jnp.int32),
                                        jnp.zeros((16,), jnp.int32)))

                def clr_body(d, _):
                    stale = jnp.full((16,), d, jnp.int32) >= ncv
                    cbs[pl.ds(d * 16, 16)] = jnp.where(
                        stale, negv, cbs[pl.ds(d * 16, 16)])
                    cbi[pl.ds(d * 16, 16)] = jnp.where(
                        stale, padi, cbi[pl.ds(d * 16, 16)])
                    return 0
                lax.fori_loop(0, DCAND, clr_body, 0)
                return ncv, lo2

            def wave_body(w, carry):
                cnt_v, tau_c = carry
                cnt_v, tau_c = lax.cond(
                    _max16(cnt_v) > DCAND - 40, reselect,
                    lambda cv, t: (cv, t), cnt_v, tau_c)
                pltpu.async_copy(
                    scores_hbm.at[cidv.at[pl.ds(w * WAVE, WAVE)]],
                    wave, sem).wait()

                def chunk_body(i, cnt2_v):
                    p = w * WAVE + i
                    for jj in range(8):
                        s = wave[i, pl.ds(jj * 16, 16)]
                        m = (s >= tau_c) & (cnt2_v < DCAND - 1)
                        enc = (p * 128 + jj * 16) + iota
                        plsc.store_scatter(cbs, [cnt2_v * 16 + iota], s,
                                           mask=m)
                        plsc.store_scatter(cbi, [cnt2_v * 16 + iota], enc,
                                           mask=m)
                        cnt2_v = cnt2_v + _ones_where(m)
                    return cnt2_v
                cnt_v = lax.fori_loop(0, WAVE, chunk_body, cnt_v)
                return (cnt_v, tau_c)

            lax.fori_loop(0, nwav, wave_body,
                          (jnp.zeros((16,), jnp.int32), tau))

            def dec_body(d, _):
                enc = cbi[pl.ds(d * 16, 16)]
                hole = enc >= PADIDX
                p = jnp.minimum(enc >> 7, jnp.int32(DC * 16 - 1))
                cid = plsc.load_gather(cidv, [p])
                gidx = (cid - r * NCHUNK) * 128 + (enc & 127)
                cbi[pl.ds(d * 16, 16)] = jnp.where(hole, padi, gidx)
                return 0
            lax.fori_loop(0, DCAND, dec_body, 0)

            pltpu.sync_copy(cbs, outs_hbm.at[r])
            pltpu.sync_copy(cbi, outi_hbm.at[r])
            return 0

        lax.fori_loop(0, rpw, per_row, 0)

    return body(scores2d, maxes)


MB = 256


def _merge_body(s_ref, i_ref, os_ref, oi_ref):
    s = s_ref[...]
    ci = i_ref[...]
    big = jnp.int32(0x7FFFFFFF)
    for it in range(100):
        m = jnp.max(s, axis=1)
        is_max = s == m[:, None]
        chosen = jnp.min(jnp.where(is_max, ci, big), axis=1)
        os_ref[:, it] = m
        oi_ref[:, it] = chosen
        s = jnp.where(ci == chosen[:, None], NEG, s)


def _final_topk(cs, ci):
    nrows = cs.shape[0]
    return pl.pallas_call(
        _merge_body,
        grid=(nrows // MB,),
        in_specs=[
            pl.BlockSpec((MB, CAP), lambda qb: (qb, 0)),
            pl.BlockSpec((MB, CAP), lambda qb: (qb, 0)),
        ],
        out_specs=[
            pl.BlockSpec((MB, 100), lambda qb: (qb, 0)),
            pl.BlockSpec((MB, 100), lambda qb: (qb, 0)),
        ],
        out_shape=[
            jax.ShapeDtypeStruct((nrows, 100), jnp.float32),
            jax.ShapeDtypeStruct((nrows, 100), jnp.int32),
        ],
    )(cs, ci)


NSLICE = 8


def kernel(query_embedding, movie_tag_embeddings, k):
    query_embedding = _normalize(query_embedding)
    movie_tag_embeddings = _normalize(movie_tag_embeddings)
    kpad = jnp.pad(movie_tag_embeddings, ((0, KPAD - NKEYS), (0, 0)))
    rs = NROWS // NSLICE
    out_s, out_i = [], []
    for i in range(NSLICE):
        q = lax.slice_in_dim(query_embedding, i * rs, (i + 1) * rs)
        scores, maxes = _scores_and_maxes(q, kpad)
        cs, ci = _sc_topk_filter(
            scores.reshape(rs * NCHUNK, 128), maxes.reshape(rs, NCHUNK), rs)
        s, ix = _final_topk(cs, ci)
        out_s.append(s)
        out_i.append(ix)
    return jnp.concatenate(out_s), jnp.concatenate(out_i)


# speedup vs baseline: 1.2080x; 1.0181x over previous
import functools

import jax
import jax.numpy as jnp
from jax import lax
from jax.experimental import pallas as pl
from jax.experimental.pallas import tpu as pltpu
from jax.experimental.pallas import tpu_sc as plsc

QB = 256
KBLK = 2048
NKB = 49
KPAD = KBLK * NKB
NKEYS = 100000
NROWS = 4096
NCHUNK = KPAD // 128
CHUNKV = NCHUNK // 16
CAP = 1024
WAVE = 64
TRIG = CAP - 128 - 16
NEG = -3.0e38
PADIDX = (1 << 30)


def _normalize(x):
    norm = jnp.sqrt(jnp.sum(x * x, axis=-1, keepdims=True))
    return x / jnp.maximum(norm, 1e-12)


def _mm_body(q_ref, k_ref, out_ref, mx_ref):
    kb = pl.program_id(1)
    qn = q_ref[...]
    kn = k_ref[...]
    s = jax.lax.dot_general(
        qn, kn, (((1,), (1,)), ((), ())),
        preferred_element_type=jnp.float32)
    col = kb * KBLK + jax.lax.broadcasted_iota(jnp.int32, (QB, KBLK), 1)
    s = jnp.where(col < NKEYS, s, NEG)
    out_ref[...] = s
    mx_ref[:, kb, :] = jnp.max(s.reshape(QB, KBLK // 128, 128), axis=2)


def _scores_and_maxes(q, kpad):
    nqb = q.shape[0] // QB
    return pl.pallas_call(
        _mm_body,
        grid=(nqb, NKB),
        in_specs=[
            pl.BlockSpec((QB, 128), lambda qb, kb: (qb, 0)),
            pl.BlockSpec((KBLK, 128), lambda qb, kb: (kb, 0)),
        ],
        out_specs=[
            pl.BlockSpec((QB, KBLK), lambda qb, kb: (qb, kb)),
            pl.BlockSpec((QB, NKB, KBLK // 128), lambda qb, kb: (qb, 0, 0)),
        ],
        out_shape=[
            jax.ShapeDtypeStruct((q.shape[0], KPAD), jnp.float32),
            jax.ShapeDtypeStruct((q.shape[0], NKB, KBLK // 128), jnp.float32),
        ],
        compiler_params=pltpu.CompilerParams(
            dimension_semantics=("arbitrary", "arbitrary")),
    )(q, kpad)


FLO = -1.001
FHI = 1.001
DC = 52
DCAND = 64
NBS = 20


def _iota16():
    return lax.iota(jnp.int32, 16)


def _sum16(v):
    acc = v[0]
    for l in range(1, 16):
        acc = acc + v[l]
    return acc


def _max16(v):
    acc = v[0]
    for l in range(1, 16):
        acc = jnp.maximum(acc, v[l])
    return acc


def _ones_where(m):
    return jnp.where(m, jnp.full((16,), 1, jnp.int32),
                     jnp.full((16,), 0, jnp.int32))


def _sc_topk_filter(scores2d, maxes, nrows):
    info = plsc.get_sparse_core_info()
    nc, ns = info.num_cores, info.num_subcores
    nw = nc * ns
    rpw = nrows // nw

    @functools.partial(
        pl.kernel,
        out_type=(jax.ShapeDtypeStruct((nrows, CAP), jnp.float32),
                  jax.ShapeDtypeStruct((nrows, CAP), jnp.int32)),
        mesh=plsc.VectorSubcoreMesh(core_axis_name="c", subcore_axis_name="s"),
        scratch_types=[
            pltpu.VMEM((NCHUNK,), jnp.float32),
            pltpu.VMEM((DC * 16,), jnp.int32),
            pltpu.VMEM((WAVE, 128), jnp.float32),
            pltpu.VMEM((CAP,), jnp.float32),
            pltpu.VMEM((CAP,), jnp.int32),
            pltpu.SemaphoreType.DMA,
        ],
        compiler_params=pltpu.CompilerParams(needs_layout_passes=False),
    )
    def body(scores_hbm, maxes_hbm, outs_hbm, outi_hbm,
             maxv, cidv, wave, cbs, cbi, sem):
        wid = lax.axis_index("s") * nc + lax.axis_index("c")
        row0 = wid * rpw
        iota = _iota16()

        def per_row(j, _):
            r = row0 + j
            pltpu.sync_copy(maxes_hbm.at[r], maxv)

            def count_maxv(th):
                def cbody(c, acc):
                    v = maxv[pl.ds(c * 16, 16)]
                    return acc + _ones_where(v >= th)
                return _sum16(lax.fori_loop(0, CHUNKV, cbody,
                                            jnp.zeros((16,), jnp.int32)))

            def bs_body(t, lohi):
                lo, hi = lohi
                mid = (lo + hi) * jnp.float32(0.5)
                big = count_maxv(mid) >= 100
                return (jnp.where(big, mid, lo), jnp.where(big, hi, mid))
            tau, _ = lax.fori_loop(
                0, NBS, bs_body, (jnp.float32(FLO), jnp.float32(FHI)))

            padcid = jnp.full((16,), r * NCHUNK + NCHUNK - 1, jnp.int32)
            def fillc_body(t, _):
                cidv[pl.ds(t * 16, 16)] = padcid
                return 0
            lax.fori_loop(0, DC, fillc_body, 0)

            def csc_body(c, ccnt_v):
                m = maxv[pl.ds(c * 16, 16)] >= tau
                gid = (r * NCHUNK + c * 16) + iota
                plsc.store_scatter(cidv, [ccnt_v * 16 + iota], gid, mask=m)
                return ccnt_v + _ones_where(m)
            ccnt_v = lax.fori_loop(0, CHUNKV, csc_body,
                                   jnp.zeros((16,), jnp.int32))
            maxfill = _max16(ccnt_v)
            nwav = (maxfill * 16 + (WAVE - 1)) // WAVE

            negv = jnp.full((16,), NEG, jnp.float32)
            padi = jnp.full((16,), PADIDX, jnp.int32)
            def init_body(t, _):
                cbs[pl.ds(t * 16, 16)] = negv
                cbi[pl.ds(t * 16, 16)] = padi
                return 0
            lax.fori_loop(0, DCAND, init_body, 0)

            def reselect(cnt_v, tau_c):
                def count_cand(th):
                    def cbody(d, acc):
                        v = cbs[pl.ds(d * 16, 16)]
                        return acc + _ones_where(v >= th)
                    return _sum16(lax.fori_loop(0, DCAND, cbody,
                                                jnp.zeros((16,), jnp.int32)))

                def rs_bs(t, lohi):
                    lo2, hi2 = lohi
                    mid = (lo2 + hi2) * jnp.float32(0.5)
                    big = count_cand(mid) >= 100
                    return (jnp.where(big, mid, lo2),
                            jnp.where(big, hi2, mid))
                lo2, hi2 = lax.fori_loop(
                    0, 45, rs_bs, (tau_c, jnp.float32(FHI)))
                g = count_cand(hi2)
                quota = jnp.int32(100) - g

                def comp_body(d, carry):
                    ncv, bandc = carry
                    v = cbs[pl.ds(d * 16, 16)]
                    ivv = cbi[pl.ds(d * 16, 16)]
                    m_hi = v >= hi2
                    m_band = (v >= lo2) & (v < hi2) & (bandc < quota)
                    m = (m_hi | m_band) & (ncv < DCAND - 8)
                    plsc.store_scatter(cbs, [ncv * 16 + iota], v, mask=m)
                    plsc.store_scatter(cbi, [ncv * 16 + iota], ivv, mask=m)
                    return (ncv + _ones_where(m),
                            bandc + _ones_where(m_band))
                ncv, _ = lax.fori_loop(0, DCAND, comp_body,
                                       (jnp.zeros((16,), jnp.int32),
                                        jnp.zeros((16,), jnp.int32)))

                def clr_body(d, _):
                    stale = jnp.full((16,), d, jnp.int32) >= ncv
                    cbs[pl.ds(d * 16, 16)] = jnp.where(
                        stale, negv, cbs[pl.ds(d * 16, 16)])
                    cbi[pl.ds(d * 16, 16)] = jnp.where(
                        stale, padi, cbi[pl.ds(d * 16, 16)])
                    return 0
                lax.fori_loop(0, DCAND, clr_body, 0)
                return ncv, lo2

            def wave_body(w, carry):
                cnt_v, tau_c = carry
                cnt_v, tau_c = lax.cond(
                    _max16(cnt_v) > DCAND - 40, reselect,
                    lambda cv, t: (cv, t), cnt_v, tau_c)
                pltpu.async_copy(
                    scores_hbm.at[cidv.at[pl.ds(w * WAVE, WAVE)]],
                    wave, sem).wait()

                def chunk_body(i, cnt2_v):
                    p = w * WAVE + i
                    for jj in range(8):
                        s = wave[i, pl.ds(jj * 16, 16)]
                        m = (s >= tau_c) & (cnt2_v < DCAND - 1)
                        enc = (p * 128 + jj * 16) + iota
                        plsc.store_scatter(cbs, [cnt2_v * 16 + iota], s,
                                           mask=m)
                        plsc.store_scatter(cbi, [cnt2_v * 16 + iota], enc,
                                           mask=m)
                        cnt2_v = cnt2_v + _ones_where(m)
                    return cnt2_v
                cnt_v = lax.fori_loop(0, WAVE, chunk_body, cnt_v)
                return (cnt_v, tau_c)

            lax.fori_loop(0, nwav, wave_body,
                          (jnp.zeros((16,), jnp.int32), tau))

            def dec_body(d, _):
                enc = cbi[pl.ds(d * 16, 16)]
                hole = enc >= PADIDX
                p = jnp.minimum(enc >> 7, jnp.int32(DC * 16 - 1))
                cid = plsc.load_gather(cidv, [p])
                gidx = (cid - r * NCHUNK) * 128 + (enc & 127)
                cbi[pl.ds(d * 16, 16)] = jnp.where(hole, padi, gidx)
                return 0
            lax.fori_loop(0, DCAND, dec_body, 0)

            pltpu.sync_copy(cbs, outs_hbm.at[r])
            pltpu.sync_copy(cbi, outi_hbm.at[r])
            return 0

        lax.fori_loop(0, rpw, per_row, 0)

    return body(scores2d, maxes)


MB = 256


def _merge_body(s_ref, i_ref, os_ref, oi_ref):
    s = s_ref[...]
    ci = i_ref[...]
    big = jnp.int32(0x7FFFFFFF)
    for it in range(100):
        m = jnp.max(s, axis=1)
        is_max = s == m[:, None]
        chosen = jnp.min(jnp.where(is_max, ci, big), axis=1)
        os_ref[:, it] = m
        oi_ref[:, it] = chosen
        s = jnp.where(ci == chosen[:, None], NEG, s)


def _final_topk(cs, ci):
    nrows = cs.shape[0]
    return pl.pallas_call(
        _merge_body,
        grid=(nrows // MB,),
        in_specs=[
            pl.BlockSpec((MB, CAP), lambda qb: (qb, 0)),
            pl.BlockSpec((MB, CAP), lambda qb: (qb, 0)),
        ],
        out_specs=[
            pl.BlockSpec((MB, 100), lambda qb: (qb, 0)),
            pl.BlockSpec((MB, 100), lambda qb: (qb, 0)),
        ],
        out_shape=[
            jax.ShapeDtypeStruct((nrows, 100), jnp.float32),
            jax.ShapeDtypeStruct((nrows, 100), jnp.int32),
        ],
    )(cs, ci)


NSLICE = 16


def kernel(query_embedding, movie_tag_embeddings, k):
    query_embedding = _normalize(query_embedding)
    movie_tag_embeddings = _normalize(movie_tag_embeddings)
    kpad = jnp.pad(movie_tag_embeddings, ((0, KPAD - NKEYS), (0, 0)))
    rs = NROWS // NSLICE
    out_s, out_i = [], []
    for i in range(NSLICE):
        q = lax.slice_in_dim(query_embedding, i * rs, (i + 1) * rs)
        scores, maxes = _scores_and_maxes(q, kpad)
        cs, ci = _sc_topk_filter(
            scores.reshape(rs * NCHUNK, 128), maxes.reshape(rs, NCHUNK), rs)
        s, ix = _final_topk(cs, ci)
        out_s.append(s)
        out_i.append(ix)
    return jnp.concatenate(out_s), jnp.concatenate(out_i)
